# baseline (speedup 1.0000x reference)
import functools

import jax
import jax.numpy as jnp
from jax import lax
from jax.experimental import pallas as pl
from jax.experimental.pallas import tpu as pltpu
from jax.experimental.pallas import tpu_sc as plsc


def _tc_build_table(embT):
    D, V = embT.shape
    CB = 1024
    nv = pl.cdiv(V, CB)

    def tr(in_ref, o_ref):
        t = jnp.transpose(in_ref[...], (1, 0))
        o_ref[...] = jnp.concatenate([t, jnp.zeros_like(t)], axis=1)

    return pl.pallas_call(
        tr,
        grid=(nv,),
        in_specs=[pl.BlockSpec((D, CB), lambda v: (0, v))],
        out_specs=pl.BlockSpec((CB, 2 * D), lambda v: (v, 0)),
        out_shape=jax.ShapeDtypeStruct((V, 2 * D), jnp.float32),
    )(embT)


def _sc_pool(x, table):
    B, H = x.shape
    V, DP = table.shape
    D = DP // 2
    info = plsc.get_sparse_core_info()
    NC, NS = info.num_cores, info.num_subcores
    NW = NC * NS
    b_per_w = B // NW
    n_dreg = D // 16
    H0 = min(128, H)
    H1 = H - H0

    mesh = plsc.VectorSubcoreMesh(core_axis_name="c", subcore_axis_name="s")

    @functools.partial(
        pl.kernel,
        mesh=mesh,
        out_type=jax.ShapeDtypeStruct((B, D), jnp.float32),
        scratch_types=[
            pltpu.VMEM((b_per_w, H), jnp.int32),
            pltpu.VMEM((4, H, DP), jnp.float32),
            pltpu.VMEM((b_per_w, D), jnp.float32),
            pltpu.SemaphoreType.DMA((4,)),
        ],
        compiler_params=pltpu.CompilerParams(use_tc_tiling_on_sc=True),
    )
    def k(x_hbm, tab_hbm, out_hbm, idx_v, rows_v, out_v, sems):
        wid = lax.axis_index("s") * NC + lax.axis_index("c")
        base = wid * b_per_w
        scale = jnp.float32(1.0 / H)

        pltpu.sync_copy(x_hbm.at[pl.ds(base, b_per_w)], idx_v)

        def gather(r, buf):
            return (
                pltpu.async_copy(
                    tab_hbm.at[idx_v.at[r, pl.ds(0, H0)]],
                    rows_v.at[buf, pl.ds(0, H0)],
                    sems.at[buf],
                ),
                pltpu.async_copy(
                    tab_hbm.at[idx_v.at[r, pl.ds(H0, H1)]],
                    rows_v.at[buf, pl.ds(H0, H1)],
                    sems.at[buf],
                ),
            )

        NBUF = 4
        pending = {}
        for r0 in range(min(NBUF, b_per_w)):
            pending[r0] = gather(r0, r0)
        for r in range(b_per_w):
            buf = r % NBUF
            for cp in pending.pop(r):
                cp.wait()

            def inner(j, accs):
                for u in range(4):
                    accs = tuple(
                        accs[d] + rows_v[buf, 4 * j + u, pl.ds(d * 16, 16)]
                        for d in range(n_dreg)
                    )
                return accs

            accs = lax.fori_loop(
                0, H // 4, inner,
                tuple(jnp.zeros((16,), jnp.float32) for _ in range(n_dreg)),
            )
            if r + NBUF < b_per_w:
                pending[r + NBUF] = gather(r + NBUF, buf)
            for d in range(n_dreg):
                out_v[r, pl.ds(d * 16, 16)] = accs[d] * scale

        pltpu.sync_copy(out_v, out_hbm.at[pl.ds(base, b_per_w)])

    return k(x, table)


def _tc_matmul_T(WT, m, b2):
    D, V = WT.shape
    B = m.shape[0]
    VB = 4096
    nv = pl.cdiv(V, VB)

    def mm(wt_ref, m_ref, b_ref, o_ref):
        o_ref[...] = (
            lax.dot_general(
                wt_ref[...], m_ref[...],
                (((0,), (1,)), ((), ())),
                preferred_element_type=jnp.float32,
            )
            + b_ref[...].T
        )

    return pl.pallas_call(
        mm,
        grid=(nv,),
        in_specs=[
            pl.BlockSpec((D, VB), lambda v: (0, v)),
            pl.BlockSpec((B, D), lambda v: (0, 0)),
            pl.BlockSpec((1, VB), lambda v: (0, v)),
        ],
        out_specs=pl.BlockSpec((VB, B), lambda v: (v, 0)),
        out_shape=jax.ShapeDtypeStruct((V, B), jnp.float32),
    )(WT, m, b2)


def kernel(x, emb_table, W, b):
    xi = jnp.maximum(x.astype(jnp.int32), 0)
    V = W.shape[0]
    b2 = b.reshape(1, V)
    table = _tc_build_table(emb_table.T)
    m = _sc_pool(xi, table)
    outT = _tc_matmul_T(W.T, m, b2)
    return outT.T


# speedup vs baseline: 1.1639x; 1.1639x over previous
import functools

import jax
import jax.numpy as jnp
from jax import lax
from jax.experimental import pallas as pl
from jax.experimental.pallas import tpu as pltpu
from jax.experimental.pallas import tpu_sc as plsc


def _tc_build_table(embT):
    D, V = embT.shape
    CB = 4096
    nv = pl.cdiv(V, CB)

    def tr(in_ref, o_ref):
        t = jnp.transpose(in_ref[...], (1, 0))
        o_ref[...] = jnp.concatenate([t, jnp.zeros_like(t)], axis=1)

    return pl.pallas_call(
        tr,
        grid=(nv,),
        in_specs=[pl.BlockSpec((D, CB), lambda v: (0, v))],
        out_specs=pl.BlockSpec((CB, 2 * D), lambda v: (v, 0)),
        out_shape=jax.ShapeDtypeStruct((V, 2 * D), jnp.float32),
    )(embT)


def _sc_pool(x, table):
    B, H = x.shape
    V, DP = table.shape
    D = DP // 2
    info = plsc.get_sparse_core_info()
    NC, NS = info.num_cores, info.num_subcores
    NW = NC * NS
    b_per_w = B // NW
    n_dreg = D // 16
    H0 = min(128, H)
    H1 = H - H0

    mesh = plsc.VectorSubcoreMesh(core_axis_name="c", subcore_axis_name="s")

    @functools.partial(
        pl.kernel,
        mesh=mesh,
        out_type=jax.ShapeDtypeStruct((B, D), jnp.float32),
        scratch_types=[
            pltpu.VMEM((b_per_w, H), jnp.int32),
            pltpu.VMEM((4, H, DP), jnp.float32),
            pltpu.VMEM((b_per_w, D), jnp.float32),
            pltpu.SemaphoreType.DMA((4,)),
        ],
        compiler_params=pltpu.CompilerParams(use_tc_tiling_on_sc=True),
    )
    def k(x_hbm, tab_hbm, out_hbm, idx_v, rows_v, out_v, sems):
        wid = lax.axis_index("s") * NC + lax.axis_index("c")
        base = wid * b_per_w
        scale = jnp.float32(1.0 / H)

        pltpu.sync_copy(x_hbm.at[pl.ds(base, b_per_w)], idx_v)

        def gather(r, buf):
            return (
                pltpu.async_copy(
                    tab_hbm.at[idx_v.at[r, pl.ds(0, H0)]],
                    rows_v.at[buf, pl.ds(0, H0)],
                    sems.at[buf],
                ),
                pltpu.async_copy(
                    tab_hbm.at[idx_v.at[r, pl.ds(H0, H1)]],
                    rows_v.at[buf, pl.ds(H0, H1)],
                    sems.at[buf],
                ),
            )

        NBUF = 4
        pending = {}
        for r0 in range(min(NBUF, b_per_w)):
            pending[r0] = gather(r0, r0)
        for r in range(b_per_w):
            buf = r % NBUF
            for cp in pending.pop(r):
                cp.wait()

            def inner(j, accs):
                for u in range(4):
                    accs = tuple(
                        accs[d] + rows_v[buf, 4 * j + u, pl.ds(d * 16, 16)]
                        for d in range(n_dreg)
                    )
                return accs

            accs = lax.fori_loop(
                0, H // 4, inner,
                tuple(jnp.zeros((16,), jnp.float32) for _ in range(n_dreg)),
            )
            if r + NBUF < b_per_w:
                pending[r + NBUF] = gather(r + NBUF, buf)
            for d in range(n_dreg):
                out_v[r, pl.ds(d * 16, 16)] = accs[d] * scale

        pltpu.sync_copy(out_v, out_hbm.at[pl.ds(base, b_per_w)])

    return k(x, table)


def _tc_matmul_T(WT, m, b2):
    D, V = WT.shape
    B = m.shape[0]
    VB = 4096
    nv = pl.cdiv(V, VB)

    def mm(wt_ref, m_ref, b_ref, o_ref):
        o_ref[...] = (
            lax.dot_general(
                wt_ref[...], m_ref[...],
                (((0,), (1,)), ((), ())),
                preferred_element_type=jnp.float32,
            )
            + b_ref[...].T
        )

    return pl.pallas_call(
        mm,
        grid=(nv,),
        in_specs=[
            pl.BlockSpec((D, VB), lambda v: (0, v)),
            pl.BlockSpec((B, D), lambda v: (0, 0)),
            pl.BlockSpec((1, VB), lambda v: (0, v)),
        ],
        out_specs=pl.BlockSpec((VB, B), lambda v: (v, 0)),
        out_shape=jax.ShapeDtypeStruct((V, B), jnp.float32),
    )(WT, m, b2)


def kernel(x, emb_table, W, b):
    xi = jnp.maximum(x.astype(jnp.int32), 0)
    V = W.shape[0]
    b2 = b.reshape(1, V)
    table = _tc_build_table(emb_table.T)
    m = _sc_pool(xi, table)
    outT = _tc_matmul_T(W.T, m, b2)
    return outT.T


# speedup vs baseline: 1.2617x; 1.0840x over previous
import functools

import jax
import jax.numpy as jnp
from jax import lax
from jax.experimental import pallas as pl
from jax.experimental.pallas import tpu as pltpu
from jax.experimental.pallas import tpu_sc as plsc


_TCB = 4096


def _tc_build_table(embT):
    D, V = embT.shape
    CB = _TCB
    nv = pl.cdiv(V, CB)

    def tr(in_ref, o_ref):
        t = jnp.transpose(in_ref[...], (1, 0))
        o_ref[...] = jnp.concatenate([t[: CB // 2], t[CB // 2 :]], axis=1)

    return pl.pallas_call(
        tr,
        grid=(nv,),
        in_specs=[pl.BlockSpec((D, CB), lambda v: (0, v))],
        out_specs=pl.BlockSpec((CB // 2, 2 * D), lambda v: (v, 0)),
        out_shape=jax.ShapeDtypeStruct((nv * CB // 2, 2 * D), jnp.float32),
    )(embT)


def _remap_idx(v):
    base = (v >> 12) << 12
    r = v & (_TCB - 1)
    half = _TCB // 2
    return base + jnp.where(r < half, r << 1, ((r - half) << 1) + 1)


def _sc_pool(x, table):
    B, H = x.shape
    V, D = table.shape
    info = plsc.get_sparse_core_info()
    NC, NS = info.num_cores, info.num_subcores
    NW = NC * NS
    b_per_w = B // NW
    n_dreg = D // 16
    H0 = min(128, H)
    H1 = H - H0

    mesh = plsc.VectorSubcoreMesh(core_axis_name="c", subcore_axis_name="s")

    @functools.partial(
        pl.kernel,
        mesh=mesh,
        out_type=jax.ShapeDtypeStruct((B, D), jnp.float32),
        scratch_types=[
            pltpu.VMEM((b_per_w, H), jnp.int32),
            pltpu.VMEM((4, H, D), jnp.float32),
            pltpu.VMEM((b_per_w, D), jnp.float32),
            pltpu.SemaphoreType.DMA((4,)),
        ],
        compiler_params=pltpu.CompilerParams(use_tc_tiling_on_sc=False),
    )
    def k(x_hbm, tab_hbm, out_hbm, idx_v, rows_v, out_v, sems):
        wid = lax.axis_index("s") * NC + lax.axis_index("c")
        base = wid * b_per_w
        scale = jnp.float32(1.0 / H)

        pltpu.sync_copy(x_hbm.at[pl.ds(base, b_per_w)], idx_v)

        def gather(r, buf):
            return (
                pltpu.async_copy(
                    tab_hbm.at[idx_v.at[r, pl.ds(0, H0)]],
                    rows_v.at[buf, pl.ds(0, H0)],
                    sems.at[buf],
                ),
                pltpu.async_copy(
                    tab_hbm.at[idx_v.at[r, pl.ds(H0, H1)]],
                    rows_v.at[buf, pl.ds(H0, H1)],
                    sems.at[buf],
                ),
            )

        NBUF = 4
        pending = {}
        for r0 in range(min(NBUF, b_per_w)):
            pending[r0] = gather(r0, r0)
        for r in range(b_per_w):
            buf = r % NBUF
            for cp in pending.pop(r):
                cp.wait()

            def inner(j, accs):
                for u in range(4):
                    accs = tuple(
                        accs[d] + rows_v[buf, 4 * j + u, pl.ds(d * 16, 16)]
                        for d in range(n_dreg)
                    )
                return accs

            accs = lax.fori_loop(
                0, H // 4, inner,
                tuple(jnp.zeros((16,), jnp.float32) for _ in range(n_dreg)),
            )
            if r + NBUF < b_per_w:
                pending[r + NBUF] = gather(r + NBUF, buf)
            for d in range(n_dreg):
                out_v[r, pl.ds(d * 16, 16)] = accs[d] * scale

        pltpu.sync_copy(out_v, out_hbm.at[pl.ds(base, b_per_w)])

    return k(x, table)


def _tc_matmul_T(WT, m, b2):
    D, V = WT.shape
    B = m.shape[0]
    VB = 4096
    nv = pl.cdiv(V, VB)

    def mm(wt_ref, m_ref, b_ref, o_ref):
        o_ref[...] = (
            lax.dot_general(
                wt_ref[...], m_ref[...],
                (((0,), (1,)), ((), ())),
                preferred_element_type=jnp.float32,
            )
            + b_ref[...].T
        )

    return pl.pallas_call(
        mm,
        grid=(nv,),
        in_specs=[
            pl.BlockSpec((D, VB), lambda v: (0, v)),
            pl.BlockSpec((B, D), lambda v: (0, 0)),
            pl.BlockSpec((1, VB), lambda v: (0, v)),
        ],
        out_specs=pl.BlockSpec((VB, B), lambda v: (v, 0)),
        out_shape=jax.ShapeDtypeStruct((V, B), jnp.float32),
    )(WT, m, b2)


def kernel(x, emb_table, W, b):
    xi = _remap_idx(jnp.maximum(x.astype(jnp.int32), 0))
    V = W.shape[0]
    b2 = b.reshape(1, V)
    D = emb_table.shape[1]
    packed = _tc_build_table(emb_table.T)
    table = packed.reshape(2 * packed.shape[0], D)
    m = _sc_pool(xi, table)
    outT = _tc_matmul_T(W.T, m, b2)
    return outT.T
